# Initial kernel scaffold; baseline (speedup 1.0000x reference)
import jax
import jax.numpy as jnp
from jax.experimental import pallas as pl


def kernel(x, edge_index, W1, a_src1, a_dst1, b1, W2, a_src2, a_dst2, b2):
    raise NotImplementedError("write your pallas kernel here")


# speedup vs baseline: 23.3758x; 23.3758x over previous
import functools

import jax
import jax.numpy as jnp
from jax import lax
from jax.experimental import pallas as pl
from jax.experimental.pallas import tpu as pltpu
from jax.experimental.pallas import tpu_sc as plsc

N = 10000
NP = 10240
D = 128
WIDE = 144
E = 320000
NT = 32
EPT = 10240
EP = NT * EPT
CH = 128
NCH = EPT // CH
RPT = NP // 16
BN = 1024
GRID = NP // BN


def _tc_transform(x, W, asv, adv):
    def body(x_ref, w_ref, asv_ref, adv_ref, hp_ref, asad_ref, m_ref, mx):
        i = pl.program_id(0)
        h = jnp.dot(x_ref[...], w_ref[...], preferred_element_type=jnp.float32)
        a_s = jnp.dot(h, asv_ref[...], preferred_element_type=jnp.float32)
        a_d = jnp.dot(h, adv_ref[...], preferred_element_type=jnp.float32)
        hp_ref[:, :D] = h
        col = lax.broadcasted_iota(jnp.int32, (BN, WIDE - D), 1)
        hp_ref[:, D:] = jnp.where(col == 0, 1.0, 0.0)
        asad_ref[0, :] = a_s[:, 0]
        asad_ref[1, :] = a_d[:, 0]

        @pl.when(i == 0)
        def _():
            mx[0] = jnp.float32(-3e38)
            mx[1] = jnp.float32(-3e38)

        mx[0] = jnp.maximum(mx[0], jnp.max(a_s))
        mx[1] = jnp.maximum(mx[1], jnp.max(a_d))

        @pl.when(i == GRID - 1)
        def _():
            m_ref[0, 0] = jnp.maximum(mx[0] + mx[1], 0.0)

    return pl.pallas_call(
        body,
        grid=(GRID,),
        in_specs=[
            pl.BlockSpec((BN, D), lambda i: (i, 0)),
            pl.BlockSpec((D, D), lambda i: (0, 0)),
            pl.BlockSpec((D, 1), lambda i: (0, 0)),
            pl.BlockSpec((D, 1), lambda i: (0, 0)),
        ],
        out_specs=[
            pl.BlockSpec((BN, WIDE), lambda i: (i, 0)),
            pl.BlockSpec((2, BN), lambda i: (0, i)),
            pl.BlockSpec((1, 1), lambda i: (0, 0), memory_space=pltpu.SMEM),
        ],
        out_shape=[
            jax.ShapeDtypeStruct((NP, WIDE), jnp.float32),
            jax.ShapeDtypeStruct((2, NP), jnp.float32),
            jax.ShapeDtypeStruct((1, 1), jnp.float32),
        ],
        scratch_shapes=[pltpu.SMEM((2,), jnp.float32)],
    )(x, W, asv, adv)


def _merge_block(nums_ref, hp_ref, asad_ref, m_ref, b_ref):
    num = nums_ref[0, :, :D] + nums_ref[1, :, :D]
    den = nums_ref[0, :, D:D + 1] + nums_ref[1, :, D:D + 1]
    h = hp_ref[:, :D]
    a_s = asad_ref[0, :][:, None]
    a_d = asad_ref[1, :][:, None]
    m = m_ref[0, 0]
    e = a_s + a_d
    e = jnp.where(e >= 0.0, e, 0.2 * e)
    exl = jnp.exp(e - m)
    out = (num + exl * h) / (den + exl + 1e-16) + b_ref[0, :]
    return jnp.where(out > 0.0, out, jnp.exp(jnp.minimum(out, 0.0)) - 1.0)


def _tc_norm_transform(nums, hp, asad, m, b, W, asv, adv):
    def body(nums_ref, hp_ref, asad_ref, m_ref, b_ref, w_ref, asv_ref,
             adv_ref, hp2_ref, asad2_ref, m2_ref, mx):
        i = pl.program_id(0)
        x2 = _merge_block(nums_ref, hp_ref, asad_ref, m_ref, b_ref)
        h = jnp.dot(x2, w_ref[...], preferred_element_type=jnp.float32)
        a_s = jnp.dot(h, asv_ref[...], preferred_element_type=jnp.float32)
        a_d = jnp.dot(h, adv_ref[...], preferred_element_type=jnp.float32)
        hp2_ref[:, :D] = h
        col = lax.broadcasted_iota(jnp.int32, (BN, WIDE - D), 1)
        hp2_ref[:, D:] = jnp.where(col == 0, 1.0, 0.0)
        asad2_ref[0, :] = a_s[:, 0]
        asad2_ref[1, :] = a_d[:, 0]

        @pl.when(i == 0)
        def _():
            mx[0] = jnp.float32(-3e38)
            mx[1] = jnp.float32(-3e38)

        mx[0] = jnp.maximum(mx[0], jnp.max(a_s))
        mx[1] = jnp.maximum(mx[1], jnp.max(a_d))

        @pl.when(i == GRID - 1)
        def _():
            m2_ref[0, 0] = jnp.maximum(mx[0] + mx[1], 0.0)

    return pl.pallas_call(
        body,
        grid=(GRID,),
        in_specs=[
            pl.BlockSpec((2, BN, WIDE), lambda i: (0, i, 0)),
            pl.BlockSpec((BN, WIDE), lambda i: (i, 0)),
            pl.BlockSpec((2, BN), lambda i: (0, i)),
            pl.BlockSpec((1, 1), lambda i: (0, 0), memory_space=pltpu.SMEM),
            pl.BlockSpec((1, D), lambda i: (0, 0)),
            pl.BlockSpec((D, D), lambda i: (0, 0)),
            pl.BlockSpec((D, 1), lambda i: (0, 0)),
            pl.BlockSpec((D, 1), lambda i: (0, 0)),
        ],
        out_specs=[
            pl.BlockSpec((BN, WIDE), lambda i: (i, 0)),
            pl.BlockSpec((2, BN), lambda i: (0, i)),
            pl.BlockSpec((1, 1), lambda i: (0, 0), memory_space=pltpu.SMEM),
        ],
        out_shape=[
            jax.ShapeDtypeStruct((NP, WIDE), jnp.float32),
            jax.ShapeDtypeStruct((2, NP), jnp.float32),
            jax.ShapeDtypeStruct((1, 1), jnp.float32),
        ],
        scratch_shapes=[pltpu.SMEM((2,), jnp.float32)],
    )(nums, hp, asad, m, b, W, asv, adv)


def _tc_norm_final(nums, hp, asad, m, b):
    def body(nums_ref, hp_ref, asad_ref, m_ref, b_ref, out_ref):
        out_ref[...] = _merge_block(nums_ref, hp_ref, asad_ref, m_ref, b_ref)

    return pl.pallas_call(
        body,
        grid=(GRID,),
        in_specs=[
            pl.BlockSpec((2, BN, WIDE), lambda i: (0, i, 0)),
            pl.BlockSpec((BN, WIDE), lambda i: (i, 0)),
            pl.BlockSpec((2, BN), lambda i: (0, i)),
            pl.BlockSpec((1, 1), lambda i: (0, 0), memory_space=pltpu.SMEM),
            pl.BlockSpec((1, D), lambda i: (0, 0)),
        ],
        out_specs=pl.BlockSpec((BN, D), lambda i: (i, 0)),
        out_shape=jax.ShapeDtypeStruct((NP, D), jnp.float32),
    )(nums, hp, asad, m, b)


def _sc_aggregate(src, dst, asv, adv, m16, hp):
    mesh = plsc.VectorSubcoreMesh(core_axis_name="c", subcore_axis_name="s")

    @functools.partial(
        pl.kernel,
        out_type=jax.ShapeDtypeStruct((2, NP, WIDE), jnp.float32),
        mesh=mesh,
        compiler_params=pltpu.CompilerParams(
            needs_layout_passes=False, use_tc_tiling_on_sc=False),
        scratch_types=[
            pltpu.VMEM_SHARED((NP, WIDE), jnp.float32),
            pltpu.VMEM((1, CH), jnp.int32),
            pltpu.VMEM((1, CH), jnp.int32),
            pltpu.VMEM((CH,), jnp.float32),
            pltpu.VMEM((CH,), jnp.float32),
            pltpu.VMEM((CH,), jnp.float32),
            pltpu.VMEM((CH, WIDE), jnp.float32),
            pltpu.VMEM((16,), jnp.float32),
            pltpu.SemaphoreType.DMA,
            pltpu.SemaphoreType.DMA,
            pltpu.SemaphoreType.DMA,
        ],
    )
    def k(src_hbm, dst_hbm, asv_hbm, adv_hbm, m_hbm, hp_hbm, nums_hbm,
          acc_sh, src_v, dst_v, asg_v, adg_v, exv_v, rows_v, m_v,
          sem1, sem2, sem3):
        c = lax.axis_index("c")
        s = lax.axis_index("s")

        def zrow(r, carry):
            for v in range(WIDE // 16):
                rows_v[r, pl.ds(v * 16, 16)] = jnp.zeros((16,), jnp.float32)
            return carry

        lax.fori_loop(0, CH, zrow, 0)
        for b in range(RPT // CH):
            pltpu.sync_copy(rows_v, acc_sh.at[pl.ds(s * RPT + b * CH, CH)])
        plsc.subcore_barrier()

        pltpu.sync_copy(m_hbm, m_v)
        mm = m_v[...]
        tile_base = (c * 16 + s) * EPT

        def chunk(kk, carry):
            e0 = pl.multiple_of(tile_base + kk * CH, CH)
            pltpu.sync_copy(src_hbm.at[pl.ds(e0, CH)], src_v.at[0])
            pltpu.sync_copy(dst_hbm.at[pl.ds(e0, CH)], dst_v.at[0])
            cp_rows = pltpu.async_copy(hp_hbm.at[src_v.at[0]], rows_v, sem3)
            cp_as = pltpu.async_copy(asv_hbm.at[src_v.at[0]], asg_v, sem1)
            cp_ad = pltpu.async_copy(adv_hbm.at[dst_v.at[0]], adg_v, sem2)
            cp_as.wait()
            cp_ad.wait()
            for v in range(CH // 16):
                e = asg_v[pl.ds(v * 16, 16)] + adg_v[pl.ds(v * 16, 16)]
                e = jnp.where(e >= 0.0, e, e * 0.2)
                exv_v[pl.ds(v * 16, 16)] = jnp.exp(e - mm)
            cp_rows.wait()

            def scale(r, carry2):
                w = plsc.load_gather(exv_v, [jnp.broadcast_to(r, (16,))])
                for v in range(WIDE // 16):
                    rows_v[r, pl.ds(v * 16, 16)] = (
                        rows_v[r, pl.ds(v * 16, 16)] * w)
                return carry2

            lax.fori_loop(0, CH, scale, 0)
            pltpu.sync_copy(rows_v, acc_sh.at[dst_v.at[0]], add=True)
            return carry

        lax.fori_loop(0, NCH, chunk, 0)
        plsc.subcore_barrier()

        for b in range(RPT // CH):
            r0 = s * RPT + b * CH
            pltpu.sync_copy(acc_sh.at[pl.ds(r0, CH)], rows_v)
            pltpu.sync_copy(rows_v, nums_hbm.at[c, pl.ds(r0, CH)])

    return k(src, dst, asv, adv, m16, hp)


def kernel(x, edge_index, W1, a_src1, a_dst1, b1, W2, a_src2, a_dst2, b2):
    xp = jnp.pad(x, ((0, NP - N), (0, 0)))
    pad = jnp.arange(EP - E, dtype=jnp.int32)
    src = jnp.concatenate([edge_index[0], pad % N])
    dst = jnp.concatenate([edge_index[1], N + pad % (NP - N)])

    asv1 = a_src1.reshape(D, 1)
    adv1 = a_dst1.reshape(D, 1)
    asv2 = a_src2.reshape(D, 1)
    adv2 = a_dst2.reshape(D, 1)

    hp1, asad1, m1 = _tc_transform(xp, W1, asv1, adv1)
    m1_16 = jnp.broadcast_to(m1.reshape(1), (16,))
    nums1 = _sc_aggregate(src, dst, asad1[0], asad1[1], m1_16, hp1)

    hp2, asad2, m2 = _tc_norm_transform(
        nums1, hp1, asad1, m1, b1.reshape(1, D), W2, asv2, adv2)
    m2_16 = jnp.broadcast_to(m2.reshape(1), (16,))
    nums2 = _sc_aggregate(src, dst, asad2[0], asad2[1], m2_16, hp2)

    out = _tc_norm_final(nums2, hp2, asad2, m2, b2.reshape(1, D))
    return out[:N]


# speedup vs baseline: 31.3671x; 1.3419x over previous
import functools

import jax
import jax.numpy as jnp
from jax import lax
from jax.experimental import pallas as pl
from jax.experimental.pallas import tpu as pltpu
from jax.experimental.pallas import tpu_sc as plsc

N = 10000
NP = 10240
D = 128
WIDE = 144
E = 320000
NT = 32
EPT = 10240
EP = NT * EPT
CH = 128
NCH = EPT // CH
RPT = NP // 16
BN = 1024
GRID = NP // BN


def _tc_transform(x, W, asv, adv):
    def body(x_ref, w_ref, asv_ref, adv_ref, hp_ref, asad_ref, m_ref, mx):
        i = pl.program_id(0)
        h = jnp.dot(x_ref[...], w_ref[...], preferred_element_type=jnp.float32)
        a_s = jnp.dot(h, asv_ref[...], preferred_element_type=jnp.float32)
        a_d = jnp.dot(h, adv_ref[...], preferred_element_type=jnp.float32)
        hp_ref[:, :D] = h
        col = lax.broadcasted_iota(jnp.int32, (BN, WIDE - D), 1)
        hp_ref[:, D:] = jnp.where(col == 0, 1.0, 0.0)
        asad_ref[0, :] = a_s[:, 0]
        asad_ref[1, :] = a_d[:, 0]

        @pl.when(i == 0)
        def _():
            mx[0] = jnp.float32(-3e38)
            mx[1] = jnp.float32(-3e38)

        mx[0] = jnp.maximum(mx[0], jnp.max(a_s))
        mx[1] = jnp.maximum(mx[1], jnp.max(a_d))

        @pl.when(i == GRID - 1)
        def _():
            m_ref[0, 0] = jnp.maximum(mx[0] + mx[1], 0.0)

    return pl.pallas_call(
        body,
        grid=(GRID,),
        in_specs=[
            pl.BlockSpec((BN, D), lambda i: (i, 0)),
            pl.BlockSpec((D, D), lambda i: (0, 0)),
            pl.BlockSpec((D, 1), lambda i: (0, 0)),
            pl.BlockSpec((D, 1), lambda i: (0, 0)),
        ],
        out_specs=[
            pl.BlockSpec((BN, WIDE), lambda i: (i, 0)),
            pl.BlockSpec((2, BN), lambda i: (0, i)),
            pl.BlockSpec((1, 1), lambda i: (0, 0), memory_space=pltpu.SMEM),
        ],
        out_shape=[
            jax.ShapeDtypeStruct((NP, WIDE), jnp.float32),
            jax.ShapeDtypeStruct((2, NP), jnp.float32),
            jax.ShapeDtypeStruct((1, 1), jnp.float32),
        ],
        scratch_shapes=[pltpu.SMEM((2,), jnp.float32)],
    )(x, W, asv, adv)


def _merge_block(nums_ref, hp_ref, asad_ref, m_ref, b_ref):
    num = nums_ref[0, :, :D] + nums_ref[1, :, :D]
    den = nums_ref[0, :, D:D + 1] + nums_ref[1, :, D:D + 1]
    h = hp_ref[:, :D]
    a_s = asad_ref[0, :][:, None]
    a_d = asad_ref[1, :][:, None]
    m = m_ref[0, 0]
    e = a_s + a_d
    e = jnp.where(e >= 0.0, e, 0.2 * e)
    exl = jnp.exp(e - m)
    out = (num + exl * h) / (den + exl + 1e-16) + b_ref[0, :]
    return jnp.where(out > 0.0, out, jnp.exp(jnp.minimum(out, 0.0)) - 1.0)


def _tc_norm_transform(nums, hp, asad, m, b, W, asv, adv):
    def body(nums_ref, hp_ref, asad_ref, m_ref, b_ref, w_ref, asv_ref,
             adv_ref, hp2_ref, asad2_ref, m2_ref, mx):
        i = pl.program_id(0)
        x2 = _merge_block(nums_ref, hp_ref, asad_ref, m_ref, b_ref)
        h = jnp.dot(x2, w_ref[...], preferred_element_type=jnp.float32)
        a_s = jnp.dot(h, asv_ref[...], preferred_element_type=jnp.float32)
        a_d = jnp.dot(h, adv_ref[...], preferred_element_type=jnp.float32)
        hp2_ref[:, :D] = h
        col = lax.broadcasted_iota(jnp.int32, (BN, WIDE - D), 1)
        hp2_ref[:, D:] = jnp.where(col == 0, 1.0, 0.0)
        asad2_ref[0, :] = a_s[:, 0]
        asad2_ref[1, :] = a_d[:, 0]

        @pl.when(i == 0)
        def _():
            mx[0] = jnp.float32(-3e38)
            mx[1] = jnp.float32(-3e38)

        mx[0] = jnp.maximum(mx[0], jnp.max(a_s))
        mx[1] = jnp.maximum(mx[1], jnp.max(a_d))

        @pl.when(i == GRID - 1)
        def _():
            m2_ref[0, 0] = jnp.maximum(mx[0] + mx[1], 0.0)

    return pl.pallas_call(
        body,
        grid=(GRID,),
        in_specs=[
            pl.BlockSpec((2, BN, WIDE), lambda i: (0, i, 0)),
            pl.BlockSpec((BN, WIDE), lambda i: (i, 0)),
            pl.BlockSpec((2, BN), lambda i: (0, i)),
            pl.BlockSpec((1, 1), lambda i: (0, 0), memory_space=pltpu.SMEM),
            pl.BlockSpec((1, D), lambda i: (0, 0)),
            pl.BlockSpec((D, D), lambda i: (0, 0)),
            pl.BlockSpec((D, 1), lambda i: (0, 0)),
            pl.BlockSpec((D, 1), lambda i: (0, 0)),
        ],
        out_specs=[
            pl.BlockSpec((BN, WIDE), lambda i: (i, 0)),
            pl.BlockSpec((2, BN), lambda i: (0, i)),
            pl.BlockSpec((1, 1), lambda i: (0, 0), memory_space=pltpu.SMEM),
        ],
        out_shape=[
            jax.ShapeDtypeStruct((NP, WIDE), jnp.float32),
            jax.ShapeDtypeStruct((2, NP), jnp.float32),
            jax.ShapeDtypeStruct((1, 1), jnp.float32),
        ],
        scratch_shapes=[pltpu.SMEM((2,), jnp.float32)],
    )(nums, hp, asad, m, b, W, asv, adv)


def _tc_norm_final(nums, hp, asad, m, b):
    def body(nums_ref, hp_ref, asad_ref, m_ref, b_ref, out_ref):
        out_ref[...] = _merge_block(nums_ref, hp_ref, asad_ref, m_ref, b_ref)

    return pl.pallas_call(
        body,
        grid=(GRID,),
        in_specs=[
            pl.BlockSpec((2, BN, WIDE), lambda i: (0, i, 0)),
            pl.BlockSpec((BN, WIDE), lambda i: (i, 0)),
            pl.BlockSpec((2, BN), lambda i: (0, i)),
            pl.BlockSpec((1, 1), lambda i: (0, 0), memory_space=pltpu.SMEM),
            pl.BlockSpec((1, D), lambda i: (0, 0)),
        ],
        out_specs=pl.BlockSpec((BN, D), lambda i: (i, 0)),
        out_shape=jax.ShapeDtypeStruct((NP, D), jnp.float32),
    )(nums, hp, asad, m, b)


def _sc_aggregate(src, dst, asv, adv, m16, hp):
    mesh = plsc.VectorSubcoreMesh(core_axis_name="c", subcore_axis_name="s")

    @functools.partial(
        pl.kernel,
        out_type=jax.ShapeDtypeStruct((2, NP, WIDE), jnp.float32),
        mesh=mesh,
        compiler_params=pltpu.CompilerParams(
            needs_layout_passes=False, use_tc_tiling_on_sc=False),
        scratch_types=[
            pltpu.VMEM_SHARED((NP, WIDE), jnp.float32),
            pltpu.VMEM((1, CH), jnp.int32),
            pltpu.VMEM((1, CH), jnp.int32),
            pltpu.VMEM((CH,), jnp.float32),
            pltpu.VMEM((CH,), jnp.float32),
            pltpu.VMEM((CH, WIDE), jnp.float32),
            pltpu.VMEM((1, CH), jnp.int32),
            pltpu.VMEM((1, CH), jnp.int32),
            pltpu.VMEM((CH,), jnp.float32),
            pltpu.VMEM((CH,), jnp.float32),
            pltpu.VMEM((CH, WIDE), jnp.float32),
            pltpu.VMEM((CH,), jnp.float32),
            pltpu.VMEM((16,), jnp.float32),
            pltpu.SemaphoreType.DMA,
            pltpu.SemaphoreType.DMA,
            pltpu.SemaphoreType.DMA,
            pltpu.SemaphoreType.DMA,
            pltpu.SemaphoreType.DMA,
            pltpu.SemaphoreType.DMA,
        ],
    )
    def k(src_hbm, dst_hbm, asv_hbm, adv_hbm, m_hbm, hp_hbm, nums_hbm,
          acc_sh, src0, dst0, asg0, adg0, rows0,
          src1, dst1, asg1, adg1, rows1, exv_v, m_v,
          gsem0, asem0, ssem0, gsem1, asem1, ssem1):
        c = lax.axis_index("c")
        s = lax.axis_index("s")
        bufs = [
            (src0, dst0, asg0, adg0, rows0, gsem0, asem0, ssem0),
            (src1, dst1, asg1, adg1, rows1, gsem1, asem1, ssem1),
        ]

        def zrow(r, carry):
            for v in range(WIDE // 16):
                rows0[r, pl.ds(v * 16, 16)] = jnp.zeros((16,), jnp.float32)
            return carry

        lax.fori_loop(0, CH, zrow, 0)
        for b in range(RPT // CH):
            pltpu.sync_copy(rows0, acc_sh.at[pl.ds(s * RPT + b * CH, CH)])
        plsc.subcore_barrier()

        pltpu.sync_copy(m_hbm, m_v)
        mm = m_v[...]
        tile_base = (c * 16 + s) * EPT

        def issue(kk, buf):
            srcb, dstb, asgb, adgb, rowsb, gsem, asem, _ = buf
            e0 = pl.multiple_of(tile_base + kk * CH, CH)
            pltpu.sync_copy(src_hbm.at[pl.ds(e0, CH)], srcb.at[0])
            pltpu.sync_copy(dst_hbm.at[pl.ds(e0, CH)], dstb.at[0])
            pltpu.async_copy(hp_hbm.at[srcb.at[0]], rowsb, gsem)
            pltpu.async_copy(asv_hbm.at[srcb.at[0]], asgb, asem)
            pltpu.async_copy(adv_hbm.at[dstb.at[0]], adgb, asem)

        issue(0, bufs[0])

        def pair(t, carry):
            for b in range(2):
                kk = 2 * t + b
                srcb, dstb, asgb, adgb, rowsb, gsem, asem, ssem = bufs[b]
                osrc, odst, oasg, oadg, orows, ogsem, oasem, ossem = \
                    bufs[1 - b]

                @pl.when(kk >= 1)
                def _():
                    pltpu.make_async_copy(
                        orows, acc_sh.at[odst.at[0]], ossem).wait()

                @pl.when(kk + 1 < NCH)
                def _():
                    issue(kk + 1, bufs[1 - b])

                pltpu.make_async_copy(asv_hbm.at[srcb.at[0]], asgb,
                                      asem).wait()
                pltpu.make_async_copy(adv_hbm.at[dstb.at[0]], adgb,
                                      asem).wait()
                for v in range(CH // 16):
                    e = asgb[pl.ds(v * 16, 16)] + adgb[pl.ds(v * 16, 16)]
                    e = jnp.where(e >= 0.0, e, e * 0.2)
                    exv_v[pl.ds(v * 16, 16)] = jnp.exp(e - mm)

                pltpu.make_async_copy(hp_hbm.at[srcb.at[0]], rowsb,
                                      gsem).wait()

                def scale(r, carry2):
                    w = plsc.load_gather(exv_v, [jnp.broadcast_to(r, (16,))])
                    for v in range(WIDE // 16):
                        rowsb[r, pl.ds(v * 16, 16)] = (
                            rowsb[r, pl.ds(v * 16, 16)] * w)
                    return carry2

                lax.fori_loop(0, CH, scale, 0)
                pltpu.async_copy(rowsb, acc_sh.at[dstb.at[0]], ssem,
                                 add=True)
            return carry

        lax.fori_loop(0, NCH // 2, pair, 0)
        pltpu.make_async_copy(rows1, acc_sh.at[dst1.at[0]], ssem1).wait()
        plsc.subcore_barrier()

        for b in range(RPT // CH):
            r0 = s * RPT + b * CH
            pltpu.sync_copy(acc_sh.at[pl.ds(r0, CH)], rows0)
            pltpu.sync_copy(rows0, nums_hbm.at[c, pl.ds(r0, CH)])

    return k(src, dst, asv, adv, m16, hp)


def kernel(x, edge_index, W1, a_src1, a_dst1, b1, W2, a_src2, a_dst2, b2):
    xp = jnp.pad(x, ((0, NP - N), (0, 0)))
    pad = jnp.arange(EP - E, dtype=jnp.int32)
    src = jnp.concatenate([edge_index[0], pad % N])
    dst = jnp.concatenate([edge_index[1], N + pad % (NP - N)])

    asv1 = a_src1.reshape(D, 1)
    adv1 = a_dst1.reshape(D, 1)
    asv2 = a_src2.reshape(D, 1)
    adv2 = a_dst2.reshape(D, 1)

    hp1, asad1, m1 = _tc_transform(xp, W1, asv1, adv1)
    m1_16 = jnp.broadcast_to(m1.reshape(1), (16,))
    nums1 = _sc_aggregate(src, dst, asad1[0], asad1[1], m1_16, hp1)

    hp2, asad2, m2 = _tc_norm_transform(
        nums1, hp1, asad1, m1, b1.reshape(1, D), W2, asv2, adv2)
    m2_16 = jnp.broadcast_to(m2.reshape(1), (16,))
    nums2 = _sc_aggregate(src, dst, asad2[0], asad2[1], m2_16, hp2)

    out = _tc_norm_final(nums2, hp2, asad2, m2, b2.reshape(1, D))
    return out[:N]


# speedup vs baseline: 36.1061x; 1.1511x over previous
import functools

import jax
import jax.numpy as jnp
from jax import lax
from jax.experimental import pallas as pl
from jax.experimental.pallas import tpu as pltpu
from jax.experimental.pallas import tpu_sc as plsc

N = 10000
NP = 10240
D = 128
WIDE = 144
E = 320000
NT = 32
EPT = 10240
EP = NT * EPT
CH = 128
NCH = EPT // CH
RPT = NP // 16
BN = 1024
GRID = NP // BN


def _tc_transform(x, W, asv, adv):
    def body(x_ref, w_ref, asv_ref, adv_ref, hp_ref, asad_ref, m_ref, mx):
        i = pl.program_id(0)
        h = jnp.dot(x_ref[...], w_ref[...], preferred_element_type=jnp.float32)
        a_s = jnp.dot(h, asv_ref[...], preferred_element_type=jnp.float32)
        a_d = jnp.dot(h, adv_ref[...], preferred_element_type=jnp.float32)
        hp_ref[:, :D] = h
        col = lax.broadcasted_iota(jnp.int32, (BN, WIDE - D), 1)
        hp_ref[:, D:] = jnp.where(col == 0, 1.0, 0.0)
        asad_ref[0, :] = a_s[:, 0]
        asad_ref[1, :] = a_d[:, 0]

        @pl.when(i == 0)
        def _():
            mx[0] = jnp.float32(-3e38)
            mx[1] = jnp.float32(-3e38)

        mx[0] = jnp.maximum(mx[0], jnp.max(a_s))
        mx[1] = jnp.maximum(mx[1], jnp.max(a_d))

        @pl.when(i == GRID - 1)
        def _():
            m_ref[0, 0] = jnp.maximum(mx[0] + mx[1], 0.0)

    return pl.pallas_call(
        body,
        grid=(GRID,),
        in_specs=[
            pl.BlockSpec((BN, D), lambda i: (i, 0)),
            pl.BlockSpec((D, D), lambda i: (0, 0)),
            pl.BlockSpec((D, 1), lambda i: (0, 0)),
            pl.BlockSpec((D, 1), lambda i: (0, 0)),
        ],
        out_specs=[
            pl.BlockSpec((BN, WIDE), lambda i: (i, 0)),
            pl.BlockSpec((2, BN), lambda i: (0, i)),
            pl.BlockSpec((1, 1), lambda i: (0, 0), memory_space=pltpu.SMEM),
        ],
        out_shape=[
            jax.ShapeDtypeStruct((NP, WIDE), jnp.float32),
            jax.ShapeDtypeStruct((2, NP), jnp.float32),
            jax.ShapeDtypeStruct((1, 1), jnp.float32),
        ],
        scratch_shapes=[pltpu.SMEM((2,), jnp.float32)],
    )(x, W, asv, adv)


def _merge_block(nums_ref, hp_ref, asad_ref, m_ref, b_ref):
    num = nums_ref[0, :, :D] + nums_ref[1, :, :D]
    den = nums_ref[0, :, D:D + 1] + nums_ref[1, :, D:D + 1]
    h = hp_ref[:, :D]
    a_s = asad_ref[0, :][:, None]
    a_d = asad_ref[1, :][:, None]
    m = m_ref[0, 0]
    e = a_s + a_d
    e = jnp.where(e >= 0.0, e, 0.2 * e)
    exl = jnp.exp(e - m)
    out = (num + exl * h) / (den + exl + 1e-16) + b_ref[0, :]
    return jnp.where(out > 0.0, out, jnp.exp(jnp.minimum(out, 0.0)) - 1.0)


def _tc_norm_transform(nums, hp, asad, m, b, W, asv, adv):
    def body(nums_ref, hp_ref, asad_ref, m_ref, b_ref, w_ref, asv_ref,
             adv_ref, hp2_ref, asad2_ref, m2_ref, mx):
        i = pl.program_id(0)
        x2 = _merge_block(nums_ref, hp_ref, asad_ref, m_ref, b_ref)
        h = jnp.dot(x2, w_ref[...], preferred_element_type=jnp.float32)
        a_s = jnp.dot(h, asv_ref[...], preferred_element_type=jnp.float32)
        a_d = jnp.dot(h, adv_ref[...], preferred_element_type=jnp.float32)
        hp2_ref[:, :D] = h
        col = lax.broadcasted_iota(jnp.int32, (BN, WIDE - D), 1)
        hp2_ref[:, D:] = jnp.where(col == 0, 1.0, 0.0)
        asad2_ref[0, :] = a_s[:, 0]
        asad2_ref[1, :] = a_d[:, 0]

        @pl.when(i == 0)
        def _():
            mx[0] = jnp.float32(-3e38)
            mx[1] = jnp.float32(-3e38)

        mx[0] = jnp.maximum(mx[0], jnp.max(a_s))
        mx[1] = jnp.maximum(mx[1], jnp.max(a_d))

        @pl.when(i == GRID - 1)
        def _():
            m2_ref[0, 0] = jnp.maximum(mx[0] + mx[1], 0.0)

    return pl.pallas_call(
        body,
        grid=(GRID,),
        in_specs=[
            pl.BlockSpec((2, BN, WIDE), lambda i: (0, i, 0)),
            pl.BlockSpec((BN, WIDE), lambda i: (i, 0)),
            pl.BlockSpec((2, BN), lambda i: (0, i)),
            pl.BlockSpec((1, 1), lambda i: (0, 0), memory_space=pltpu.SMEM),
            pl.BlockSpec((1, D), lambda i: (0, 0)),
            pl.BlockSpec((D, D), lambda i: (0, 0)),
            pl.BlockSpec((D, 1), lambda i: (0, 0)),
            pl.BlockSpec((D, 1), lambda i: (0, 0)),
        ],
        out_specs=[
            pl.BlockSpec((BN, WIDE), lambda i: (i, 0)),
            pl.BlockSpec((2, BN), lambda i: (0, i)),
            pl.BlockSpec((1, 1), lambda i: (0, 0), memory_space=pltpu.SMEM),
        ],
        out_shape=[
            jax.ShapeDtypeStruct((NP, WIDE), jnp.float32),
            jax.ShapeDtypeStruct((2, NP), jnp.float32),
            jax.ShapeDtypeStruct((1, 1), jnp.float32),
        ],
        scratch_shapes=[pltpu.SMEM((2,), jnp.float32)],
    )(nums, hp, asad, m, b, W, asv, adv)


def _tc_norm_final(nums, hp, asad, m, b):
    def body(nums_ref, hp_ref, asad_ref, m_ref, b_ref, out_ref):
        out_ref[...] = _merge_block(nums_ref, hp_ref, asad_ref, m_ref, b_ref)

    return pl.pallas_call(
        body,
        grid=(GRID,),
        in_specs=[
            pl.BlockSpec((2, BN, WIDE), lambda i: (0, i, 0)),
            pl.BlockSpec((BN, WIDE), lambda i: (i, 0)),
            pl.BlockSpec((2, BN), lambda i: (0, i)),
            pl.BlockSpec((1, 1), lambda i: (0, 0), memory_space=pltpu.SMEM),
            pl.BlockSpec((1, D), lambda i: (0, 0)),
        ],
        out_specs=pl.BlockSpec((BN, D), lambda i: (i, 0)),
        out_shape=jax.ShapeDtypeStruct((NP, D), jnp.float32),
    )(nums, hp, asad, m, b)


def _sc_aggregate(src, dst, asv, adv, m16, hp):
    mesh = plsc.VectorSubcoreMesh(core_axis_name="c", subcore_axis_name="s")

    @functools.partial(
        pl.kernel,
        out_type=jax.ShapeDtypeStruct((2, NP, WIDE), jnp.float32),
        mesh=mesh,
        compiler_params=pltpu.CompilerParams(
            needs_layout_passes=False, use_tc_tiling_on_sc=False),
        scratch_types=[
            pltpu.VMEM_SHARED((NP, WIDE), jnp.float32),
            pltpu.VMEM((1, CH), jnp.int32),
            pltpu.VMEM((1, CH), jnp.int32),
            pltpu.VMEM((CH,), jnp.float32),
            pltpu.VMEM((CH,), jnp.float32),
            pltpu.VMEM((CH, WIDE), jnp.float32),
            pltpu.VMEM((1, CH), jnp.int32),
            pltpu.VMEM((1, CH), jnp.int32),
            pltpu.VMEM((CH,), jnp.float32),
            pltpu.VMEM((CH,), jnp.float32),
            pltpu.VMEM((CH, WIDE), jnp.float32),
            pltpu.VMEM((CH,), jnp.float32),
            pltpu.VMEM((16,), jnp.float32),
            pltpu.SemaphoreType.DMA,
            pltpu.SemaphoreType.DMA,
            pltpu.SemaphoreType.DMA,
            pltpu.SemaphoreType.DMA,
            pltpu.SemaphoreType.DMA,
            pltpu.SemaphoreType.DMA,
        ],
    )
    def k(src_hbm, dst_hbm, asv_hbm, adv_hbm, m_hbm, hp_hbm, nums_hbm,
          acc_sh, src0, dst0, asg0, adg0, rows0,
          src1, dst1, asg1, adg1, rows1, exv_v, m_v,
          gsem0, asem0, ssem0, gsem1, asem1, ssem1):
        c = lax.axis_index("c")
        s = lax.axis_index("s")
        bufs = [
            (src0, dst0, asg0, adg0, rows0, gsem0, asem0, ssem0),
            (src1, dst1, asg1, adg1, rows1, gsem1, asem1, ssem1),
        ]

        def zrow(r, carry):
            for v in range(WIDE // 16):
                rows0[r, pl.ds(v * 16, 16)] = jnp.zeros((16,), jnp.float32)
            return carry

        lax.fori_loop(0, CH, zrow, 0)
        for b in range(RPT // CH):
            pltpu.sync_copy(rows0, acc_sh.at[pl.ds(s * RPT + b * CH, CH)])
        plsc.subcore_barrier()

        pltpu.sync_copy(m_hbm, m_v)
        mm = m_v[...]
        tile_base = (c * 16 + s) * EPT

        def issue(kk, buf):
            srcb, dstb, asgb, adgb, rowsb, gsem, asem, _ = buf
            e0 = pl.multiple_of(tile_base + kk * CH, CH)
            pltpu.sync_copy(src_hbm.at[pl.ds(e0, CH)], srcb.at[0])
            pltpu.sync_copy(dst_hbm.at[pl.ds(e0, CH)], dstb.at[0])
            pltpu.async_copy(hp_hbm.at[srcb.at[0]], rowsb, gsem)
            pltpu.async_copy(asv_hbm.at[srcb.at[0]], asgb, asem)
            pltpu.async_copy(adv_hbm.at[dstb.at[0]], adgb, asem)

        issue(0, bufs[0])

        def pair(t, carry):
            for b in range(2):
                kk = 2 * t + b
                srcb, dstb, asgb, adgb, rowsb, gsem, asem, ssem = bufs[b]
                osrc, odst, oasg, oadg, orows, ogsem, oasem, ossem = \
                    bufs[1 - b]

                @pl.when(kk >= 1)
                def _():
                    pltpu.make_async_copy(
                        orows, acc_sh.at[odst.at[0]], ossem).wait()

                @pl.when(kk + 1 < NCH)
                def _():
                    issue(kk + 1, bufs[1 - b])

                pltpu.make_async_copy(asv_hbm.at[srcb.at[0]], asgb,
                                      asem).wait()
                pltpu.make_async_copy(adv_hbm.at[dstb.at[0]], adgb,
                                      asem).wait()
                for v in range(CH // 16):
                    e = asgb[pl.ds(v * 16, 16)] + adgb[pl.ds(v * 16, 16)]
                    e = jnp.where(e >= 0.0, e, e * 0.2)
                    exv_v[pl.ds(v * 16, 16)] = jnp.exp(e - mm)

                pltpu.make_async_copy(hp_hbm.at[srcb.at[0]], rowsb,
                                      gsem).wait()

                @plsc.parallel_loop(0, CH, step=1, unroll=4)
                def scale(r):
                    w = plsc.load_gather(exv_v, [jnp.broadcast_to(r, (16,))])
                    for v in range(WIDE // 16):
                        rowsb[r, pl.ds(v * 16, 16)] = (
                            rowsb[r, pl.ds(v * 16, 16)] * w)
                pltpu.async_copy(rowsb, acc_sh.at[dstb.at[0]], ssem,
                                 add=True)
            return carry

        lax.fori_loop(0, NCH // 2, pair, 0)
        pltpu.make_async_copy(rows1, acc_sh.at[dst1.at[0]], ssem1).wait()
        plsc.subcore_barrier()

        for b in range(RPT // CH):
            r0 = s * RPT + b * CH
            pltpu.sync_copy(acc_sh.at[pl.ds(r0, CH)], rows0)
            pltpu.sync_copy(rows0, nums_hbm.at[c, pl.ds(r0, CH)])

    return k(src, dst, asv, adv, m16, hp)


def kernel(x, edge_index, W1, a_src1, a_dst1, b1, W2, a_src2, a_dst2, b2):
    xp = jnp.pad(x, ((0, NP - N), (0, 0)))
    pad = jnp.arange(EP - E, dtype=jnp.int32)
    src = jnp.concatenate([edge_index[0], pad % N])
    dst = jnp.concatenate([edge_index[1], N + pad % (NP - N)])

    asv1 = a_src1.reshape(D, 1)
    adv1 = a_dst1.reshape(D, 1)
    asv2 = a_src2.reshape(D, 1)
    adv2 = a_dst2.reshape(D, 1)

    hp1, asad1, m1 = _tc_transform(xp, W1, asv1, adv1)
    m1_16 = jnp.broadcast_to(m1.reshape(1), (16,))
    nums1 = _sc_aggregate(src, dst, asad1[0], asad1[1], m1_16, hp1)

    hp2, asad2, m2 = _tc_norm_transform(
        nums1, hp1, asad1, m1, b1.reshape(1, D), W2, asv2, adv2)
    m2_16 = jnp.broadcast_to(m2.reshape(1), (16,))
    nums2 = _sc_aggregate(src, dst, asad2[0], asad2[1], m2_16, hp2)

    out = _tc_norm_final(nums2, hp2, asad2, m2, b2.reshape(1, D))
    return out[:N]
